# baseline (speedup 1.0000x reference)
import jax
import jax.numpy as jnp
from jax import lax
from jax.experimental import pallas as pl
from jax.experimental.pallas import tpu as pltpu
from jax.experimental.pallas import tpu_sc as plsc

N = 10000
NPAD = 10240
E = 320000
EPAD = 327680
D = 128
DH = 256
NC = 2
NS = 16
NW = NC * NS
EPW = EPAD // NW
CH = 128
NCHUNK = EPW // CH
NGROUP = 2
GCHUNK = NCHUNK // NGROUP
TCHUNK = EPAD // CH
K0 = TCHUNK // NS
AGC = 16
RSUB = NPAD // NS
DEGW = 128
BR = 1024


def _sc_deg(dst_hbm, ones_hbm, zeros_hbm, out_hbm, idx_v, ones_v, acc_sh):
    c = lax.axis_index("c")
    s = lax.axis_index("s")
    wid = c * NS + s
    pltpu.sync_copy(ones_hbm, ones_v)
    pltpu.sync_copy(zeros_hbm, acc_sh.at[pl.ds(s * RSUB, RSUB)])
    plsc.subcore_barrier()

    for h in range(NGROUP):
        pltpu.sync_copy(dst_hbm.at[pl.ds(wid * NCHUNK + h * GCHUNK, GCHUNK)], idx_v)

        def body(i, carry):
            pltpu.sync_copy(ones_v, acc_sh.at[idx_v.at[i]], add=True)
            return carry

        lax.fori_loop(0, GCHUNK, body, 0)
    plsc.subcore_barrier()
    pltpu.sync_copy(
        acc_sh.at[pl.ds(s * RSUB, RSUB)],
        out_hbm.at[c, pl.ds(s * RSUB, RSUB)],
    )


def _sc_agg(val_hbm, src_hbm, dst_hbm, zeros_hbm, out_hbm,
            src_v, dst_v, rows_a, rows_b, acc_sh, sem_a, sem_b):
    c = lax.axis_index("c")
    s = lax.axis_index("s")

    @pl.when(c == 0)
    def _core0():
        pltpu.sync_copy(zeros_hbm, acc_sh.at[pl.ds(s * RSUB, RSUB)])
        plsc.subcore_barrier()

        for h in range(K0 // AGC):
            gb = s * K0 + h * AGC
            pltpu.sync_copy(src_hbm.at[pl.ds(gb, AGC)], src_v)
            pltpu.sync_copy(dst_hbm.at[pl.ds(gb, AGC)], dst_v)
            pltpu.async_copy(val_hbm.at[src_v.at[0]], rows_a, sem_a)

            def body(k, carry2):
                i0 = 2 * k
                pltpu.async_copy(val_hbm.at[src_v.at[i0 + 1]], rows_b, sem_b)
                pltpu.make_async_copy(val_hbm.at[src_v.at[i0]], rows_a, sem_a).wait()
                pltpu.sync_copy(rows_a, acc_sh.at[dst_v.at[i0]], add=True)
                pltpu.async_copy(val_hbm.at[src_v.at[i0 + 2]], rows_a, sem_a)
                pltpu.make_async_copy(val_hbm.at[src_v.at[i0 + 1]], rows_b, sem_b).wait()
                pltpu.sync_copy(rows_b, acc_sh.at[dst_v.at[i0 + 1]], add=True)
                return carry2

            lax.fori_loop(0, AGC // 2 - 1, body, 0)
            i0 = AGC - 2
            pltpu.async_copy(val_hbm.at[src_v.at[i0 + 1]], rows_b, sem_b)
            pltpu.make_async_copy(val_hbm.at[src_v.at[i0]], rows_a, sem_a).wait()
            pltpu.sync_copy(rows_a, acc_sh.at[dst_v.at[i0]], add=True)
            pltpu.make_async_copy(val_hbm.at[src_v.at[i0 + 1]], rows_b, sem_b).wait()
            pltpu.sync_copy(rows_b, acc_sh.at[dst_v.at[i0 + 1]], add=True)
        plsc.subcore_barrier()
        pltpu.sync_copy(
            acc_sh.at[pl.ds(s * RSUB, RSUB)],
            out_hbm.at[pl.ds(s * RSUB, RSUB)],
        )


def _tc_prep(degp_ref, x_ref, d_ref, xs_ref):
    deg = degp_ref[0] + degp_ref[1]
    dv = lax.rsqrt(deg[:, 0:1] + 1.0)
    d_ref[...] = dv
    xs_ref[...] = x_ref[...] * dv


def _tc_mlp(p_ref, xs_ref, d_ref, m_ref, w0_ref, b0_ref, w1_ref, z_ref):
    agg = p_ref[...] + xs_ref[...]
    dv = d_ref[...]
    h = jnp.dot(agg * dv, w0_ref[...], preferred_element_type=jnp.float32)
    x1 = jnp.maximum(h + b0_ref[...], 0.0)
    z_ref[...] = jnp.dot(x1 * (dv * m_ref[...]), w1_ref[...],
                         preferred_element_type=jnp.float32)


def _tc_out(q_ref, z_ref, d_ref, b1_ref, o_ref):
    agg = q_ref[...] + z_ref[...]
    o_ref[...] = jnp.maximum(agg * d_ref[...] + b1_ref[...], 0.0)


def kernel(x, edge_index, W0, b0, W1, b1):
    f32 = jnp.float32
    ei = edge_index.astype(jnp.int32)
    pad_e = EPAD - E
    pad_idx = jnp.full((pad_e,), N, jnp.int32)
    src3 = jnp.concatenate([ei[0], pad_idx]).reshape(TCHUNK, CH)
    dst3 = jnp.concatenate([ei[1], pad_idx]).reshape(TCHUNK, CH)
    x_pad = jnp.concatenate([x, jnp.zeros((NPAD - N, D), f32)], axis=0)
    ones_tpl = jnp.zeros((CH, DEGW), f32).at[:, 0].set(1.0)
    zdeg = jnp.zeros((RSUB, DEGW), f32)
    zrow = jnp.zeros((RSUB, D), f32)
    maskc = (jnp.arange(NPAD) < N).astype(f32)[:, None]
    b0r = b0.reshape(1, DH)
    b1r = b1.reshape(1, D)

    mesh = plsc.VectorSubcoreMesh(core_axis_name="c", subcore_axis_name="s")
    deg_call = pl.kernel(
        _sc_deg,
        out_type=jax.ShapeDtypeStruct((NC, NPAD, DEGW), f32),
        mesh=mesh,
        scratch_types=[
            pltpu.VMEM((GCHUNK, CH), jnp.int32),
            pltpu.VMEM((CH, DEGW), f32),
            pltpu.VMEM_SHARED((NPAD, DEGW), f32),
        ],
    )
    agg_call = pl.kernel(
        _sc_agg,
        out_type=jax.ShapeDtypeStruct((NPAD, D), f32),
        mesh=mesh,
        scratch_types=[
            pltpu.VMEM((AGC, CH), jnp.int32),
            pltpu.VMEM((AGC, CH), jnp.int32),
            pltpu.VMEM((CH, D), f32),
            pltpu.VMEM((CH, D), f32),
            pltpu.VMEM_SHARED((NPAD, D), f32),
            pltpu.SemaphoreType.DMA,
            pltpu.SemaphoreType.DMA,
        ],
    )

    grid = (NPAD // BR,)
    row_spec = pl.BlockSpec((BR, D), lambda i: (i, 0))
    col_spec = pl.BlockSpec((BR, 1), lambda i: (i, 0))

    degp = deg_call(dst3, ones_tpl, zdeg)

    d_col, xs = pl.pallas_call(
        _tc_prep,
        grid=grid,
        in_specs=[pl.BlockSpec((NC, BR, DEGW), lambda i: (0, i, 0)), row_spec],
        out_specs=[col_spec, row_spec],
        out_shape=[
            jax.ShapeDtypeStruct((NPAD, 1), f32),
            jax.ShapeDtypeStruct((NPAD, D), f32),
        ],
    )(degp, x_pad)

    p = agg_call(xs, src3, dst3, zrow)

    z = pl.pallas_call(
        _tc_mlp,
        grid=grid,
        in_specs=[
            row_spec, row_spec, col_spec, col_spec,
            pl.BlockSpec((D, DH), lambda i: (0, 0)),
            pl.BlockSpec((1, DH), lambda i: (0, 0)),
            pl.BlockSpec((DH, D), lambda i: (0, 0)),
        ],
        out_specs=row_spec,
        out_shape=jax.ShapeDtypeStruct((NPAD, D), f32),
    )(p, xs, d_col, maskc, W0, b0r, W1)

    q = agg_call(z, src3, dst3, zrow)

    out = pl.pallas_call(
        _tc_out,
        grid=grid,
        in_specs=[
            row_spec, row_spec, col_spec,
            pl.BlockSpec((1, D), lambda i: (0, 0)),
        ],
        out_specs=row_spec,
        out_shape=jax.ShapeDtypeStruct((NPAD, D), f32),
    )(q, z, d_col, b1r)

    return out[:N]


# speedup vs baseline: 1.0021x; 1.0021x over previous
import jax
import jax.numpy as jnp
from jax import lax
from jax.experimental import pallas as pl
from jax.experimental.pallas import tpu as pltpu
from jax.experimental.pallas import tpu_sc as plsc

N = 10000
NPAD = 10240
E = 320000
EPAD = 327680
D = 128
DH = 256
NC = 2
NS = 16
NW = NC * NS
EPW = EPAD // NW
CH = 128
NCHUNK = EPW // CH
NGROUP = 2
GCHUNK = NCHUNK // NGROUP
TCHUNK = EPAD // CH
K0 = TCHUNK // NS
AGC = 16
RSUB = NPAD // NS
DEGW = 128
BR = 1024


def _sc_deg(dst_hbm, ones_hbm, zeros_hbm, out_hbm, idx_v, ones_v, acc_sh):
    c = lax.axis_index("c")
    s = lax.axis_index("s")
    wid = c * NS + s
    pltpu.sync_copy(ones_hbm, ones_v)
    pltpu.sync_copy(zeros_hbm, acc_sh.at[pl.ds(s * RSUB, RSUB)])
    plsc.subcore_barrier()

    for h in range(NGROUP):
        pltpu.sync_copy(dst_hbm.at[pl.ds(wid * NCHUNK + h * GCHUNK, GCHUNK)], idx_v)

        def body(i, carry):
            pltpu.sync_copy(ones_v, acc_sh.at[idx_v.at[i]], add=True)
            return carry

        lax.fori_loop(0, GCHUNK, body, 0)
    plsc.subcore_barrier()
    pltpu.sync_copy(
        acc_sh.at[pl.ds(s * RSUB, RSUB)],
        out_hbm.at[c, pl.ds(s * RSUB, RSUB)],
    )


def _sc_agg(val_hbm, src_hbm, dst_hbm, zeros_hbm, out_hbm,
            src_v, dst_v, rows_a, rows_b, acc_sh, sem_a, sem_b):
    c = lax.axis_index("c")
    s = lax.axis_index("s")

    @pl.when(c == 0)
    def _core0():
        pltpu.sync_copy(zeros_hbm, acc_sh.at[pl.ds(s * RSUB, RSUB)])
        plsc.subcore_barrier()

        def group(g, carry):
            gb = s * K0 + g * AGC
            pltpu.sync_copy(src_hbm.at[pl.ds(gb, AGC)], src_v)
            pltpu.sync_copy(dst_hbm.at[pl.ds(gb, AGC)], dst_v)
            pltpu.async_copy(val_hbm.at[src_v.at[0]], rows_a, sem_a)

            def body(k, carry2):
                i0 = 2 * k
                pltpu.async_copy(val_hbm.at[src_v.at[i0 + 1]], rows_b, sem_b)
                pltpu.make_async_copy(val_hbm.at[src_v.at[i0]], rows_a, sem_a).wait()
                pltpu.sync_copy(rows_a, acc_sh.at[dst_v.at[i0]], add=True)
                pltpu.async_copy(val_hbm.at[src_v.at[i0 + 2]], rows_a, sem_a)
                pltpu.make_async_copy(val_hbm.at[src_v.at[i0 + 1]], rows_b, sem_b).wait()
                pltpu.sync_copy(rows_b, acc_sh.at[dst_v.at[i0 + 1]], add=True)
                return carry2

            lax.fori_loop(0, AGC // 2 - 1, body, 0)
            i0 = AGC - 2
            pltpu.async_copy(val_hbm.at[src_v.at[i0 + 1]], rows_b, sem_b)
            pltpu.make_async_copy(val_hbm.at[src_v.at[i0]], rows_a, sem_a).wait()
            pltpu.sync_copy(rows_a, acc_sh.at[dst_v.at[i0]], add=True)
            pltpu.make_async_copy(val_hbm.at[src_v.at[i0 + 1]], rows_b, sem_b).wait()
            pltpu.sync_copy(rows_b, acc_sh.at[dst_v.at[i0 + 1]], add=True)
            return carry

        lax.fori_loop(0, K0 // AGC, group, 0)
        plsc.subcore_barrier()
        pltpu.sync_copy(
            acc_sh.at[pl.ds(s * RSUB, RSUB)],
            out_hbm.at[pl.ds(s * RSUB, RSUB)],
        )


def _tc_prep(degp_ref, x_ref, d_ref, xs_ref):
    deg = degp_ref[0] + degp_ref[1]
    dv = lax.rsqrt(deg[:, 0:1] + 1.0)
    d_ref[...] = dv
    xs_ref[...] = x_ref[...] * dv


def _tc_mlp(p_ref, xs_ref, d_ref, m_ref, w0_ref, b0_ref, w1_ref, z_ref):
    agg = p_ref[...] + xs_ref[...]
    dv = d_ref[...]
    h = jnp.dot(agg * dv, w0_ref[...], preferred_element_type=jnp.float32)
    x1 = jnp.maximum(h + b0_ref[...], 0.0)
    z_ref[...] = jnp.dot(x1 * (dv * m_ref[...]), w1_ref[...],
                         preferred_element_type=jnp.float32)


def _tc_out(q_ref, z_ref, d_ref, b1_ref, o_ref):
    agg = q_ref[...] + z_ref[...]
    o_ref[...] = jnp.maximum(agg * d_ref[...] + b1_ref[...], 0.0)


def kernel(x, edge_index, W0, b0, W1, b1):
    f32 = jnp.float32
    ei = edge_index.astype(jnp.int32)
    pad_e = EPAD - E
    pad_idx = jnp.full((pad_e,), N, jnp.int32)
    src3 = jnp.concatenate([ei[0], pad_idx]).reshape(TCHUNK, CH)
    dst3 = jnp.concatenate([ei[1], pad_idx]).reshape(TCHUNK, CH)
    x_pad = jnp.concatenate([x, jnp.zeros((NPAD - N, D), f32)], axis=0)
    ones_tpl = jnp.zeros((CH, DEGW), f32).at[:, 0].set(1.0)
    zdeg = jnp.zeros((RSUB, DEGW), f32)
    zrow = jnp.zeros((RSUB, D), f32)
    maskc = (jnp.arange(NPAD) < N).astype(f32)[:, None]
    b0r = b0.reshape(1, DH)
    b1r = b1.reshape(1, D)

    mesh = plsc.VectorSubcoreMesh(core_axis_name="c", subcore_axis_name="s")
    deg_call = pl.kernel(
        _sc_deg,
        out_type=jax.ShapeDtypeStruct((NC, NPAD, DEGW), f32),
        mesh=mesh,
        scratch_types=[
            pltpu.VMEM((GCHUNK, CH), jnp.int32),
            pltpu.VMEM((CH, DEGW), f32),
            pltpu.VMEM_SHARED((NPAD, DEGW), f32),
        ],
    )
    agg_call = pl.kernel(
        _sc_agg,
        out_type=jax.ShapeDtypeStruct((NPAD, D), f32),
        mesh=mesh,
        scratch_types=[
            pltpu.VMEM((AGC, CH), jnp.int32),
            pltpu.VMEM((AGC, CH), jnp.int32),
            pltpu.VMEM((CH, D), f32),
            pltpu.VMEM((CH, D), f32),
            pltpu.VMEM_SHARED((NPAD, D), f32),
            pltpu.SemaphoreType.DMA,
            pltpu.SemaphoreType.DMA,
        ],
    )

    grid = (NPAD // BR,)
    row_spec = pl.BlockSpec((BR, D), lambda i: (i, 0))
    col_spec = pl.BlockSpec((BR, 1), lambda i: (i, 0))

    degp = deg_call(dst3, ones_tpl, zdeg)

    d_col, xs = pl.pallas_call(
        _tc_prep,
        grid=grid,
        in_specs=[pl.BlockSpec((NC, BR, DEGW), lambda i: (0, i, 0)), row_spec],
        out_specs=[col_spec, row_spec],
        out_shape=[
            jax.ShapeDtypeStruct((NPAD, 1), f32),
            jax.ShapeDtypeStruct((NPAD, D), f32),
        ],
    )(degp, x_pad)

    p = agg_call(xs, src3, dst3, zrow)

    z = pl.pallas_call(
        _tc_mlp,
        grid=grid,
        in_specs=[
            row_spec, row_spec, col_spec, col_spec,
            pl.BlockSpec((D, DH), lambda i: (0, 0)),
            pl.BlockSpec((1, DH), lambda i: (0, 0)),
            pl.BlockSpec((DH, D), lambda i: (0, 0)),
        ],
        out_specs=row_spec,
        out_shape=jax.ShapeDtypeStruct((NPAD, D), f32),
    )(p, xs, d_col, maskc, W0, b0r, W1)

    q = agg_call(z, src3, dst3, zrow)

    out = pl.pallas_call(
        _tc_out,
        grid=grid,
        in_specs=[
            row_spec, row_spec, col_spec,
            pl.BlockSpec((1, D), lambda i: (0, 0)),
        ],
        out_specs=row_spec,
        out_shape=jax.ShapeDtypeStruct((NPAD, D), f32),
    )(q, z, d_col, b1r)

    return out[:N]


# speedup vs baseline: 1.0125x; 1.0104x over previous
import jax
import jax.numpy as jnp
from jax import lax
from jax.experimental import pallas as pl
from jax.experimental.pallas import tpu as pltpu
from jax.experimental.pallas import tpu_sc as plsc

N = 10000
NPAD = 10240
E = 320000
EPAD = 327680
D = 128
DH = 256
NC = 2
NS = 16
NW = NC * NS
EPW = EPAD // NW
CH = 128
NCHUNK = EPW // CH
NGROUP = 2
GCHUNK = NCHUNK // NGROUP
TCHUNK = EPAD // CH
K0 = 160
K1 = 0
AGC = 16
NTC0 = NS * K0
RSUB = NPAD // NS
DEGW = 128
BR = 1024


def _sc_deg(dst_hbm, ones_hbm, zeros_hbm, out_hbm, idx_v, ones_v, acc_sh):
    c = lax.axis_index("c")
    s = lax.axis_index("s")
    wid = c * NS + s
    pltpu.sync_copy(ones_hbm, ones_v)
    pltpu.sync_copy(zeros_hbm, acc_sh.at[pl.ds(s * RSUB, RSUB)])
    plsc.subcore_barrier()

    for h in range(NGROUP):
        pltpu.sync_copy(dst_hbm.at[pl.ds(wid * NCHUNK + h * GCHUNK, GCHUNK)], idx_v)

        def body(i, carry):
            pltpu.sync_copy(ones_v, acc_sh.at[idx_v.at[i]], add=True)
            return carry

        lax.fori_loop(0, GCHUNK, body, 0)
    plsc.subcore_barrier()
    pltpu.sync_copy(
        acc_sh.at[pl.ds(s * RSUB, RSUB)],
        out_hbm.at[c, pl.ds(s * RSUB, RSUB)],
    )


def _sc_agg(val_hbm, src_hbm, dst_hbm, zeros_hbm, out_hbm,
            src_v, dst_v, rows_a, rows_b, acc_sh, sem_a, sem_b):
    c = lax.axis_index("c")
    s = lax.axis_index("s")
    pltpu.sync_copy(zeros_hbm, acc_sh.at[pl.ds(s * RSUB, RSUB)])
    plsc.subcore_barrier()

    ngr = jnp.where(c == 0, K0 // AGC, K1 // AGC)
    cbase = jnp.where(c == 0, s * K0, NTC0 + s * K1)

    if True:
        def group(g, carry):
            gb = cbase + g * AGC
            pltpu.sync_copy(src_hbm.at[pl.ds(gb, AGC)], src_v)
            pltpu.sync_copy(dst_hbm.at[pl.ds(gb, AGC)], dst_v)
            pltpu.async_copy(val_hbm.at[src_v.at[0]], rows_a, sem_a)

            def body(k, carry2):
                i0 = 2 * k
                pltpu.async_copy(val_hbm.at[src_v.at[i0 + 1]], rows_b, sem_b)
                pltpu.make_async_copy(val_hbm.at[src_v.at[i0]], rows_a, sem_a).wait()
                pltpu.sync_copy(rows_a, acc_sh.at[dst_v.at[i0]], add=True)
                pltpu.async_copy(val_hbm.at[src_v.at[i0 + 2]], rows_a, sem_a)
                pltpu.make_async_copy(val_hbm.at[src_v.at[i0 + 1]], rows_b, sem_b).wait()
                pltpu.sync_copy(rows_b, acc_sh.at[dst_v.at[i0 + 1]], add=True)
                return carry2

            lax.fori_loop(0, AGC // 2 - 1, body, 0)
            i0 = AGC - 2
            pltpu.async_copy(val_hbm.at[src_v.at[i0 + 1]], rows_b, sem_b)
            pltpu.make_async_copy(val_hbm.at[src_v.at[i0]], rows_a, sem_a).wait()
            pltpu.sync_copy(rows_a, acc_sh.at[dst_v.at[i0]], add=True)
            pltpu.make_async_copy(val_hbm.at[src_v.at[i0 + 1]], rows_b, sem_b).wait()
            pltpu.sync_copy(rows_b, acc_sh.at[dst_v.at[i0 + 1]], add=True)
            return carry

        lax.fori_loop(0, ngr, group, 0)
        plsc.subcore_barrier()
        pltpu.sync_copy(
            acc_sh.at[pl.ds(s * RSUB, RSUB)],
            out_hbm.at[c, pl.ds(s * RSUB, RSUB)],
        )


def _tc_prep(degp_ref, x_ref, d_ref, xs_ref):
    deg = degp_ref[0] + degp_ref[1]
    dv = lax.rsqrt(deg[:, 0:1] + 1.0)
    d_ref[...] = dv
    xs_ref[...] = x_ref[...] * dv


def _tc_mlp(p_ref, xs_ref, d_ref, m_ref, w0_ref, b0_ref, w1_ref, z_ref):
    agg = p_ref[0] + p_ref[1] + xs_ref[...]
    dv = d_ref[...]
    h = jnp.dot(agg * dv, w0_ref[...], preferred_element_type=jnp.float32)
    x1 = jnp.maximum(h + b0_ref[...], 0.0)
    z_ref[...] = jnp.dot(x1 * (dv * m_ref[...]), w1_ref[...],
                         preferred_element_type=jnp.float32)


def _tc_out(q_ref, z_ref, d_ref, b1_ref, o_ref):
    agg = q_ref[0] + q_ref[1] + z_ref[...]
    o_ref[...] = jnp.maximum(agg * d_ref[...] + b1_ref[...], 0.0)


def kernel(x, edge_index, W0, b0, W1, b1):
    f32 = jnp.float32
    ei = edge_index.astype(jnp.int32)
    pad_e = EPAD - E
    pad_idx = jnp.full((pad_e,), N, jnp.int32)
    src3 = jnp.concatenate([ei[0], pad_idx]).reshape(TCHUNK, CH)
    dst3 = jnp.concatenate([ei[1], pad_idx]).reshape(TCHUNK, CH)
    x_pad = jnp.concatenate([x, jnp.zeros((NPAD - N, D), f32)], axis=0)
    ones_tpl = jnp.zeros((CH, DEGW), f32).at[:, 0].set(1.0)
    zdeg = jnp.zeros((RSUB, DEGW), f32)
    zrow = jnp.zeros((RSUB, D), f32)
    maskc = (jnp.arange(NPAD) < N).astype(f32)[:, None]
    b0r = b0.reshape(1, DH)
    b1r = b1.reshape(1, D)

    mesh = plsc.VectorSubcoreMesh(core_axis_name="c", subcore_axis_name="s")
    deg_call = pl.kernel(
        _sc_deg,
        out_type=jax.ShapeDtypeStruct((NC, NPAD, DEGW), f32),
        mesh=mesh,
        scratch_types=[
            pltpu.VMEM((GCHUNK, CH), jnp.int32),
            pltpu.VMEM((CH, DEGW), f32),
            pltpu.VMEM_SHARED((NPAD, DEGW), f32),
        ],
    )
    agg_call = pl.kernel(
        _sc_agg,
        out_type=jax.ShapeDtypeStruct((NC, NPAD, D), f32),
        mesh=mesh,
        scratch_types=[
            pltpu.VMEM((AGC, CH), jnp.int32),
            pltpu.VMEM((AGC, CH), jnp.int32),
            pltpu.VMEM((CH, D), f32),
            pltpu.VMEM((CH, D), f32),
            pltpu.VMEM_SHARED((NPAD, D), f32),
            pltpu.SemaphoreType.DMA,
            pltpu.SemaphoreType.DMA,
        ],
    )

    grid = (NPAD // BR,)
    row_spec = pl.BlockSpec((BR, D), lambda i: (i, 0))
    col_spec = pl.BlockSpec((BR, 1), lambda i: (i, 0))
    part_spec = pl.BlockSpec((NC, BR, D), lambda i: (0, i, 0))

    degp = deg_call(dst3, ones_tpl, zdeg)

    d_col, xs = pl.pallas_call(
        _tc_prep,
        grid=grid,
        in_specs=[pl.BlockSpec((NC, BR, DEGW), lambda i: (0, i, 0)), row_spec],
        out_specs=[col_spec, row_spec],
        out_shape=[
            jax.ShapeDtypeStruct((NPAD, 1), f32),
            jax.ShapeDtypeStruct((NPAD, D), f32),
        ],
    )(degp, x_pad)

    p = agg_call(xs, src3, dst3, zrow)

    z = pl.pallas_call(
        _tc_mlp,
        grid=grid,
        in_specs=[
            part_spec, row_spec, col_spec, col_spec,
            pl.BlockSpec((D, DH), lambda i: (0, 0)),
            pl.BlockSpec((1, DH), lambda i: (0, 0)),
            pl.BlockSpec((DH, D), lambda i: (0, 0)),
        ],
        out_specs=row_spec,
        out_shape=jax.ShapeDtypeStruct((NPAD, D), f32),
    )(p, xs, d_col, maskc, W0, b0r, W1)

    q = agg_call(z, src3, dst3, zrow)

    out = pl.pallas_call(
        _tc_out,
        grid=grid,
        in_specs=[
            part_spec, row_spec, col_spec,
            pl.BlockSpec((1, D), lambda i: (0, 0)),
        ],
        out_specs=row_spec,
        out_shape=jax.ShapeDtypeStruct((NPAD, D), f32),
    )(q, z, d_col, b1r)

    return out[:N]


# speedup vs baseline: 2.0878x; 2.0620x over previous
import jax
import jax.numpy as jnp
from jax import lax
from jax.experimental import pallas as pl
from jax.experimental.pallas import tpu as pltpu
from jax.experimental.pallas import tpu_sc as plsc

N = 10000
NPAD = 10240
E = 320000
EPAD = 327680
D = 128
DH = 256
NC = 2
NS = 16
NW = NC * NS
EPW = EPAD // NW
CH = 128
NCHUNK = EPW // CH
NGROUP = 2
GCHUNK = NCHUNK // NGROUP
TCHUNK = EPAD // CH
K0 = 160
K1 = 0
AGC = 16
NTC0 = NS * K0
RSUB = NPAD // NS
DEGW = 128
BR = 1024


def _sc_deg(dst_hbm, ones_hbm, zeros_hbm, out_hbm, idx_v, ones_v, acc_sh):
    c = lax.axis_index("c")
    s = lax.axis_index("s")
    wid = c * NS + s
    pltpu.sync_copy(ones_hbm, ones_v)
    pltpu.sync_copy(zeros_hbm, acc_sh.at[pl.ds(s * RSUB, RSUB)])
    plsc.subcore_barrier()

    for h in range(NGROUP):
        pltpu.sync_copy(dst_hbm.at[pl.ds(wid * NCHUNK + h * GCHUNK, GCHUNK)], idx_v)

        def body(i, carry):
            pltpu.sync_copy(ones_v, acc_sh.at[idx_v.at[i]], add=True)
            return carry

        lax.fori_loop(0, GCHUNK, body, 0)
    plsc.subcore_barrier()
    pltpu.sync_copy(
        acc_sh.at[pl.ds(s * RSUB, RSUB)],
        out_hbm.at[c, pl.ds(s * RSUB, RSUB)],
    )


def _sc_agg(val_hbm, src_hbm, dst_hbm, zeros_hbm, out_hbm,
            src_v, dst_v, rows_a, rows_b, acc_sh, sem_a, sem_b):
    c = lax.axis_index("c")
    s = lax.axis_index("s")
    pltpu.sync_copy(zeros_hbm, acc_sh.at[pl.ds(s * RSUB, RSUB)])
    plsc.subcore_barrier()

    ngr = jnp.where(c == 0, K0 // AGC, K1 // AGC)
    cbase = jnp.where(c == 0, s * K0, NTC0 + s * K1)

    if True:
        def group(g, carry):
            gb = cbase + g * AGC
            pltpu.sync_copy(src_hbm.at[pl.ds(gb, AGC)], src_v)
            pltpu.sync_copy(dst_hbm.at[pl.ds(gb, AGC)], dst_v)
            pltpu.async_copy(val_hbm.at[src_v.at[0]], rows_a, sem_a)

            def body(k, carry2):
                i0 = 2 * k
                pltpu.async_copy(val_hbm.at[src_v.at[i0 + 1]], rows_b, sem_b)
                pltpu.make_async_copy(val_hbm.at[src_v.at[i0]], rows_a, sem_a).wait()
                pltpu.sync_copy(rows_a, acc_sh.at[dst_v.at[i0]], add=True)
                pltpu.async_copy(val_hbm.at[src_v.at[i0 + 2]], rows_a, sem_a)
                pltpu.make_async_copy(val_hbm.at[src_v.at[i0 + 1]], rows_b, sem_b).wait()
                pltpu.sync_copy(rows_b, acc_sh.at[dst_v.at[i0 + 1]], add=True)
                return carry2

            lax.fori_loop(0, AGC // 2 - 1, body, 0)
            i0 = AGC - 2
            pltpu.async_copy(val_hbm.at[src_v.at[i0 + 1]], rows_b, sem_b)
            pltpu.make_async_copy(val_hbm.at[src_v.at[i0]], rows_a, sem_a).wait()
            pltpu.sync_copy(rows_a, acc_sh.at[dst_v.at[i0]], add=True)
            pltpu.make_async_copy(val_hbm.at[src_v.at[i0 + 1]], rows_b, sem_b).wait()
            pltpu.sync_copy(rows_b, acc_sh.at[dst_v.at[i0 + 1]], add=True)
            return carry

        lax.fori_loop(0, ngr, group, 0)
        plsc.subcore_barrier()
        pltpu.sync_copy(
            acc_sh.at[pl.ds(s * RSUB, RSUB)],
            out_hbm.at[c, pl.ds(s * RSUB, RSUB)],
        )


def _tc_prep(degp_ref, x_ref, d_ref, xs_ref):
    deg = degp_ref[0] + degp_ref[1]
    dv = lax.rsqrt(deg[:, 0:1] + 1.0)
    d_ref[...] = dv
    xs_ref[...] = x_ref[...] * dv


def _tc_mlp(p_ref, xs_ref, d_ref, m_ref, w0_ref, b0_ref, w1_ref, z_ref):
    agg = p_ref[0] + p_ref[1] + xs_ref[...]
    dv = d_ref[...]
    h = jnp.dot(agg * dv, w0_ref[...], preferred_element_type=jnp.float32)
    x1 = jnp.maximum(h + b0_ref[...], 0.0)
    z_ref[...] = jnp.dot(x1 * (dv * m_ref[...]), w1_ref[...],
                         preferred_element_type=jnp.float32)


def _tc_out(q_ref, z_ref, d_ref, b1_ref, o_ref):
    agg = q_ref[0] + q_ref[1] + z_ref[...]
    o_ref[...] = jnp.maximum(agg * d_ref[...] + b1_ref[...], 0.0)


def kernel(x, edge_index, W0, b0, W1, b1):
    f32 = jnp.float32
    ei = edge_index.astype(jnp.int32)
    pad_e = EPAD - E
    ar = jnp.arange(pad_e, dtype=jnp.int32)
    pad_src = ar % N
    pad_dst = N + (ar % (NPAD - N))
    src3 = jnp.concatenate([ei[0], pad_src]).reshape(TCHUNK, CH)
    dst3 = jnp.concatenate([ei[1], pad_dst]).reshape(TCHUNK, CH)
    x_pad = jnp.concatenate([x, jnp.zeros((NPAD - N, D), f32)], axis=0)
    ones_tpl = jnp.zeros((CH, DEGW), f32).at[:, 0].set(1.0)
    zdeg = jnp.zeros((RSUB, DEGW), f32)
    zrow = jnp.zeros((RSUB, D), f32)
    maskc = (jnp.arange(NPAD) < N).astype(f32)[:, None]
    b0r = b0.reshape(1, DH)
    b1r = b1.reshape(1, D)

    mesh = plsc.VectorSubcoreMesh(core_axis_name="c", subcore_axis_name="s")
    deg_call = pl.kernel(
        _sc_deg,
        out_type=jax.ShapeDtypeStruct((NC, NPAD, DEGW), f32),
        mesh=mesh,
        scratch_types=[
            pltpu.VMEM((GCHUNK, CH), jnp.int32),
            pltpu.VMEM((CH, DEGW), f32),
            pltpu.VMEM_SHARED((NPAD, DEGW), f32),
        ],
    )
    agg_call = pl.kernel(
        _sc_agg,
        out_type=jax.ShapeDtypeStruct((NC, NPAD, D), f32),
        mesh=mesh,
        scratch_types=[
            pltpu.VMEM((AGC, CH), jnp.int32),
            pltpu.VMEM((AGC, CH), jnp.int32),
            pltpu.VMEM((CH, D), f32),
            pltpu.VMEM((CH, D), f32),
            pltpu.VMEM_SHARED((NPAD, D), f32),
            pltpu.SemaphoreType.DMA,
            pltpu.SemaphoreType.DMA,
        ],
    )

    grid = (NPAD // BR,)
    row_spec = pl.BlockSpec((BR, D), lambda i: (i, 0))
    col_spec = pl.BlockSpec((BR, 1), lambda i: (i, 0))
    part_spec = pl.BlockSpec((NC, BR, D), lambda i: (0, i, 0))

    degp = deg_call(dst3, ones_tpl, zdeg)

    d_col, xs = pl.pallas_call(
        _tc_prep,
        grid=grid,
        in_specs=[pl.BlockSpec((NC, BR, DEGW), lambda i: (0, i, 0)), row_spec],
        out_specs=[col_spec, row_spec],
        out_shape=[
            jax.ShapeDtypeStruct((NPAD, 1), f32),
            jax.ShapeDtypeStruct((NPAD, D), f32),
        ],
    )(degp, x_pad)

    p = agg_call(xs, src3, dst3, zrow)

    z = pl.pallas_call(
        _tc_mlp,
        grid=grid,
        in_specs=[
            part_spec, row_spec, col_spec, col_spec,
            pl.BlockSpec((D, DH), lambda i: (0, 0)),
            pl.BlockSpec((1, DH), lambda i: (0, 0)),
            pl.BlockSpec((DH, D), lambda i: (0, 0)),
        ],
        out_specs=row_spec,
        out_shape=jax.ShapeDtypeStruct((NPAD, D), f32),
    )(p, xs, d_col, maskc, W0, b0r, W1)

    q = agg_call(z, src3, dst3, zrow)

    out = pl.pallas_call(
        _tc_out,
        grid=grid,
        in_specs=[
            part_spec, row_spec, col_spec,
            pl.BlockSpec((1, D), lambda i: (0, 0)),
        ],
        out_specs=row_spec,
        out_shape=jax.ShapeDtypeStruct((NPAD, D), f32),
    )(q, z, d_col, b1r)

    return out[:N]


# speedup vs baseline: 3.1299x; 1.4991x over previous
import jax
import jax.numpy as jnp
from jax import lax
from jax.experimental import pallas as pl
from jax.experimental.pallas import tpu as pltpu
from jax.experimental.pallas import tpu_sc as plsc

N = 10000
NPAD = 10240
E = 320000
EPAD = 327680
D = 128
DH = 256
NC = 2
NS = 16
NW = NC * NS
EPW = EPAD // NW
CH = 128
NCHUNK = EPW // CH
NGROUP = 2
GCHUNK = NCHUNK // NGROUP
TCHUNK = EPAD // CH
K0 = 80
K1 = 80
AGC = 16
NTC0 = NS * K0
RSUB = NPAD // NS
DEGW = 128
BR = 1024


def _sc_deg(dst_hbm, ones_hbm, zeros_hbm, out_hbm, idx_v, ones_v, acc_sh):
    c = lax.axis_index("c")
    s = lax.axis_index("s")
    wid = c * NS + s
    pltpu.sync_copy(ones_hbm, ones_v)
    pltpu.sync_copy(zeros_hbm, acc_sh.at[pl.ds(s * RSUB, RSUB)])
    plsc.subcore_barrier()

    for h in range(NGROUP):
        pltpu.sync_copy(dst_hbm.at[pl.ds(wid * NCHUNK + h * GCHUNK, GCHUNK)], idx_v)

        def body(i, carry):
            pltpu.sync_copy(ones_v, acc_sh.at[idx_v.at[i]], add=True)
            return carry

        lax.fori_loop(0, GCHUNK, body, 0)
    plsc.subcore_barrier()
    pltpu.sync_copy(
        acc_sh.at[pl.ds(s * RSUB, RSUB)],
        out_hbm.at[c, pl.ds(s * RSUB, RSUB)],
    )


def _sc_agg(val_hbm, src_hbm, dst_hbm, zeros_hbm, out_hbm,
            src_v, dst_v, rows_a, rows_b, acc_sh, sem_a, sem_b):
    c = lax.axis_index("c")
    s = lax.axis_index("s")
    pltpu.sync_copy(zeros_hbm, acc_sh.at[pl.ds(s * RSUB, RSUB)])
    plsc.subcore_barrier()

    ngr = jnp.where(c == 0, K0 // AGC, K1 // AGC)
    cbase = jnp.where(c == 0, s * K0, NTC0 + s * K1)

    if True:
        def group(g, carry):
            gb = cbase + g * AGC
            pltpu.sync_copy(src_hbm.at[pl.ds(gb, AGC)], src_v)
            pltpu.sync_copy(dst_hbm.at[pl.ds(gb, AGC)], dst_v)
            pltpu.async_copy(val_hbm.at[src_v.at[0]], rows_a, sem_a)

            def body(k, carry2):
                i0 = 2 * k
                pltpu.async_copy(val_hbm.at[src_v.at[i0 + 1]], rows_b, sem_b)
                pltpu.make_async_copy(val_hbm.at[src_v.at[i0]], rows_a, sem_a).wait()
                pltpu.sync_copy(rows_a, acc_sh.at[dst_v.at[i0]], add=True)
                pltpu.async_copy(val_hbm.at[src_v.at[i0 + 2]], rows_a, sem_a)
                pltpu.make_async_copy(val_hbm.at[src_v.at[i0 + 1]], rows_b, sem_b).wait()
                pltpu.sync_copy(rows_b, acc_sh.at[dst_v.at[i0 + 1]], add=True)
                return carry2

            lax.fori_loop(0, AGC // 2 - 1, body, 0)
            i0 = AGC - 2
            pltpu.async_copy(val_hbm.at[src_v.at[i0 + 1]], rows_b, sem_b)
            pltpu.make_async_copy(val_hbm.at[src_v.at[i0]], rows_a, sem_a).wait()
            pltpu.sync_copy(rows_a, acc_sh.at[dst_v.at[i0]], add=True)
            pltpu.make_async_copy(val_hbm.at[src_v.at[i0 + 1]], rows_b, sem_b).wait()
            pltpu.sync_copy(rows_b, acc_sh.at[dst_v.at[i0 + 1]], add=True)
            return carry

        lax.fori_loop(0, ngr, group, 0)
        plsc.subcore_barrier()
        pltpu.sync_copy(
            acc_sh.at[pl.ds(s * RSUB, RSUB)],
            out_hbm.at[c, pl.ds(s * RSUB, RSUB)],
        )


def _tc_prep(degp_ref, x_ref, d_ref, xs_ref):
    deg = degp_ref[0] + degp_ref[1]
    dv = lax.rsqrt(deg[:, 0:1] + 1.0)
    d_ref[...] = dv
    xs_ref[...] = x_ref[...] * dv


def _tc_mlp(p_ref, xs_ref, d_ref, m_ref, w0_ref, b0_ref, w1_ref, z_ref):
    agg = p_ref[0] + p_ref[1] + xs_ref[...]
    dv = d_ref[...]
    h = jnp.dot(agg * dv, w0_ref[...], preferred_element_type=jnp.float32)
    x1 = jnp.maximum(h + b0_ref[...], 0.0)
    z_ref[...] = jnp.dot(x1 * (dv * m_ref[...]), w1_ref[...],
                         preferred_element_type=jnp.float32)


def _tc_out(q_ref, z_ref, d_ref, b1_ref, o_ref):
    agg = q_ref[0] + q_ref[1] + z_ref[...]
    o_ref[...] = jnp.maximum(agg * d_ref[...] + b1_ref[...], 0.0)


def kernel(x, edge_index, W0, b0, W1, b1):
    f32 = jnp.float32
    ei = edge_index.astype(jnp.int32)
    pad_e = EPAD - E
    ar = jnp.arange(pad_e, dtype=jnp.int32)
    pad_src = ar % N
    pad_dst = N + (ar % (NPAD - N))
    src3 = jnp.concatenate([ei[0], pad_src]).reshape(TCHUNK, CH)
    dst3 = jnp.concatenate([ei[1], pad_dst]).reshape(TCHUNK, CH)
    x_pad = jnp.concatenate([x, jnp.zeros((NPAD - N, D), f32)], axis=0)
    ones_tpl = jnp.zeros((CH, DEGW), f32).at[:, 0].set(1.0)
    zdeg = jnp.zeros((RSUB, DEGW), f32)
    zrow = jnp.zeros((RSUB, D), f32)
    maskc = (jnp.arange(NPAD) < N).astype(f32)[:, None]
    b0r = b0.reshape(1, DH)
    b1r = b1.reshape(1, D)

    mesh = plsc.VectorSubcoreMesh(core_axis_name="c", subcore_axis_name="s")
    deg_call = pl.kernel(
        _sc_deg,
        out_type=jax.ShapeDtypeStruct((NC, NPAD, DEGW), f32),
        mesh=mesh,
        scratch_types=[
            pltpu.VMEM((GCHUNK, CH), jnp.int32),
            pltpu.VMEM((CH, DEGW), f32),
            pltpu.VMEM_SHARED((NPAD, DEGW), f32),
        ],
    )
    agg_call = pl.kernel(
        _sc_agg,
        out_type=jax.ShapeDtypeStruct((NC, NPAD, D), f32),
        mesh=mesh,
        scratch_types=[
            pltpu.VMEM((AGC, CH), jnp.int32),
            pltpu.VMEM((AGC, CH), jnp.int32),
            pltpu.VMEM((CH, D), f32),
            pltpu.VMEM((CH, D), f32),
            pltpu.VMEM_SHARED((NPAD, D), f32),
            pltpu.SemaphoreType.DMA,
            pltpu.SemaphoreType.DMA,
        ],
    )

    grid = (NPAD // BR,)
    row_spec = pl.BlockSpec((BR, D), lambda i: (i, 0))
    col_spec = pl.BlockSpec((BR, 1), lambda i: (i, 0))
    part_spec = pl.BlockSpec((NC, BR, D), lambda i: (0, i, 0))

    degp = deg_call(dst3, ones_tpl, zdeg)

    d_col, xs = pl.pallas_call(
        _tc_prep,
        grid=grid,
        in_specs=[pl.BlockSpec((NC, BR, DEGW), lambda i: (0, i, 0)), row_spec],
        out_specs=[col_spec, row_spec],
        out_shape=[
            jax.ShapeDtypeStruct((NPAD, 1), f32),
            jax.ShapeDtypeStruct((NPAD, D), f32),
        ],
    )(degp, x_pad)

    p = agg_call(xs, src3, dst3, zrow)

    z = pl.pallas_call(
        _tc_mlp,
        grid=grid,
        in_specs=[
            part_spec, row_spec, col_spec, col_spec,
            pl.BlockSpec((D, DH), lambda i: (0, 0)),
            pl.BlockSpec((1, DH), lambda i: (0, 0)),
            pl.BlockSpec((DH, D), lambda i: (0, 0)),
        ],
        out_specs=row_spec,
        out_shape=jax.ShapeDtypeStruct((NPAD, D), f32),
    )(p, xs, d_col, maskc, W0, b0r, W1)

    q = agg_call(z, src3, dst3, zrow)

    out = pl.pallas_call(
        _tc_out,
        grid=grid,
        in_specs=[
            part_spec, row_spec, col_spec,
            pl.BlockSpec((1, D), lambda i: (0, 0)),
        ],
        out_specs=row_spec,
        out_shape=jax.ShapeDtypeStruct((NPAD, D), f32),
    )(q, z, d_col, b1r)

    return out[:N]
